# Initial kernel scaffold; baseline (speedup 1.0000x reference)
import jax
import jax.numpy as jnp
from jax.experimental import pallas as pl


def kernel(features, edge_index, W1, b1, W2, b2):
    raise NotImplementedError("write your pallas kernel here")


# speedup vs baseline: 19.1855x; 19.1855x over previous
import functools

import jax
import jax.numpy as jnp
from jax import lax
from jax.experimental import pallas as pl
from jax.experimental.pallas import tpu as pltpu
from jax.experimental.pallas import tpu_sc as plsc

N = 10000
NP = 10240
D_IN = 128
H = 16
C = 7
E = 320000
CHUNK = 128
NW = 32
RPW = 80
R = RPW * NW
EP = R * CHUNK
ROWS_PER_TILE = NP // 16


def _deg_body(src_hbm, dst_hbm, deg_o_hbm, deg_i_hbm,
              idx_v, ones_v, z_v, deg_o_sp, deg_i_sp, sem):
    c = lax.axis_index("c")
    s = lax.axis_index("s")
    wid = c * 16 + s

    for i in range(CHUNK // 16):
        ones_v[pl.ds(i * 16, 16)] = jnp.ones((16,), jnp.float32)
    for i in range(ROWS_PER_TILE // 16):
        z_v[pl.ds(i * 16, 16)] = jnp.zeros((16,), jnp.float32)
    pltpu.sync_copy(z_v, deg_o_sp.at[pl.ds(s * ROWS_PER_TILE, ROWS_PER_TILE)])
    pltpu.sync_copy(z_v, deg_i_sp.at[pl.ds(s * ROWS_PER_TILE, ROWS_PER_TILE)])
    plsc.subcore_barrier()

    pltpu.sync_copy(src_hbm.at[pl.ds(wid * RPW, RPW)], idx_v)

    def add_src(j, carry):
        pltpu.sync_copy(ones_v, deg_o_sp.at[idx_v.at[j]], add=True)
        return carry

    lax.fori_loop(0, RPW, add_src, 0)

    pltpu.sync_copy(dst_hbm.at[pl.ds(wid * RPW, RPW)], idx_v)

    def add_dst(j, carry):
        pltpu.sync_copy(ones_v, deg_i_sp.at[idx_v.at[j]], add=True)
        return carry

    lax.fori_loop(0, RPW, add_dst, 0)
    plsc.subcore_barrier()

    sl = pl.ds(s * ROWS_PER_TILE, ROWS_PER_TILE)
    pltpu.sync_copy(deg_o_sp.at[sl], deg_o_hbm.at[c].at[sl])
    pltpu.sync_copy(deg_i_sp.at[sl], deg_i_hbm.at[c].at[sl])


_SC_PARAMS = pltpu.CompilerParams(use_tc_tiling_on_sc=False)

_deg_call = functools.partial(
    pl.kernel,
    out_type=(jax.ShapeDtypeStruct((2, NP), jnp.float32),
              jax.ShapeDtypeStruct((2, NP), jnp.float32)),
    mesh=plsc.VectorSubcoreMesh(core_axis_name="c", subcore_axis_name="s"),
    compiler_params=_SC_PARAMS,
    scratch_types=(
        pltpu.VMEM((RPW, CHUNK), jnp.int32),
        pltpu.VMEM((CHUNK,), jnp.float32),
        pltpu.VMEM((ROWS_PER_TILE,), jnp.float32),
        pltpu.VMEM_SHARED((NP,), jnp.float32),
        pltpu.VMEM_SHARED((NP,), jnp.float32),
        pltpu.SemaphoreType.DMA,
    ),
)(_deg_body)


def _gs_body(src_hbm, dst_hbm, h_hbm, out_hbm,
             srcv, dstv, rows_v, z_v, agg_sp, sem):
    c = lax.axis_index("c")
    s = lax.axis_index("s")
    wid = c * 16 + s

    for i in range(64):
        z_v[i, :] = jnp.zeros((16,), jnp.float32)

    def zero_tile(t, carry):
        pltpu.sync_copy(z_v, agg_sp.at[pl.ds(s * ROWS_PER_TILE + t * 64, 64)])
        return carry

    lax.fori_loop(0, ROWS_PER_TILE // 64, zero_tile, 0)
    plsc.subcore_barrier()

    pltpu.sync_copy(src_hbm.at[pl.ds(wid * RPW, RPW)], srcv)
    pltpu.sync_copy(dst_hbm.at[pl.ds(wid * RPW, RPW)], dstv)

    def chunk(j, carry):
        pltpu.async_copy(h_hbm.at[srcv.at[j]], rows_v, sem).wait()
        pltpu.sync_copy(rows_v, agg_sp.at[dstv.at[j]], add=True)
        return carry

    lax.fori_loop(0, RPW, chunk, 0)
    plsc.subcore_barrier()

    sl = pl.ds(s * ROWS_PER_TILE, ROWS_PER_TILE)
    pltpu.sync_copy(agg_sp.at[sl], out_hbm.at[c].at[sl])


_gs_call = functools.partial(
    pl.kernel,
    out_type=jax.ShapeDtypeStruct((2, NP, H), jnp.float32),
    mesh=plsc.VectorSubcoreMesh(core_axis_name="c", subcore_axis_name="s"),
    compiler_params=_SC_PARAMS,
    scratch_types=(
        pltpu.VMEM((RPW, CHUNK), jnp.int32),
        pltpu.VMEM((RPW, CHUNK), jnp.int32),
        pltpu.VMEM((CHUNK, H), jnp.float32),
        pltpu.VMEM((64, H), jnp.float32),
        pltpu.VMEM_SHARED((NP, H), jnp.float32),
        pltpu.SemaphoreType.DMA,
    ),
)(_gs_body)


_BLK = 1024
_GRID = NP // _BLK


def _k1_body(feat_ref, w1_ref, dgo_ref, dgi_ref, h1s_ref, ns_ref, nd_ref):
    deg_o = dgo_ref[0] + dgo_ref[1]
    deg_i = dgi_ref[0] + dgi_ref[1]
    ns = lax.rsqrt(jnp.maximum(deg_o, 1.0))
    nd = lax.rsqrt(jnp.maximum(deg_i, 1.0))
    h = jnp.dot(feat_ref[...], w1_ref[...], preferred_element_type=jnp.float32)
    h1s_ref[...] = h * ns[:, None]
    ns_ref[...] = ns
    nd_ref[...] = nd


def _k1(feat_p, w1, deg_o, deg_i):
    return pl.pallas_call(
        _k1_body,
        grid=(_GRID,),
        in_specs=[
            pl.BlockSpec((_BLK, D_IN), lambda i: (i, 0)),
            pl.BlockSpec((D_IN, H), lambda i: (0, 0)),
            pl.BlockSpec((2, _BLK), lambda i: (0, i)),
            pl.BlockSpec((2, _BLK), lambda i: (0, i)),
        ],
        out_specs=[
            pl.BlockSpec((_BLK, H), lambda i: (i, 0)),
            pl.BlockSpec((_BLK,), lambda i: (i,)),
            pl.BlockSpec((_BLK,), lambda i: (i,)),
        ],
        out_shape=[
            jax.ShapeDtypeStruct((NP, H), jnp.float32),
            jax.ShapeDtypeStruct((NP,), jnp.float32),
            jax.ShapeDtypeStruct((NP,), jnp.float32),
        ],
    )(feat_p, w1, deg_o, deg_i)


def _k2_body(p_ref, nd_ref, ns_ref, b1_ref, w2_ref, out_ref):
    agg = p_ref[0] + p_ref[1]
    x1 = jnp.maximum(agg * nd_ref[...][:, None] + b1_ref[...], 0.0)
    h2 = jnp.dot(x1, w2_ref[...], preferred_element_type=jnp.float32)
    out_ref[...] = h2 * ns_ref[...][:, None]


def _k2(parts, nd, ns, b1, w2p):
    return pl.pallas_call(
        _k2_body,
        grid=(_GRID,),
        in_specs=[
            pl.BlockSpec((2, _BLK, H), lambda i: (0, i, 0)),
            pl.BlockSpec((_BLK,), lambda i: (i,)),
            pl.BlockSpec((_BLK,), lambda i: (i,)),
            pl.BlockSpec((1, H), lambda i: (0, 0)),
            pl.BlockSpec((H, H), lambda i: (0, 0)),
        ],
        out_specs=pl.BlockSpec((_BLK, H), lambda i: (i, 0)),
        out_shape=jax.ShapeDtypeStruct((NP, H), jnp.float32),
    )(parts, nd, ns, b1, w2p)


def _k3_body(p_ref, nd_ref, b2_ref, out_ref):
    agg = p_ref[0] + p_ref[1]
    out_ref[...] = agg * nd_ref[...][:, None] + b2_ref[...]


def _k3(parts, nd, b2p):
    return pl.pallas_call(
        _k3_body,
        grid=(_GRID,),
        in_specs=[
            pl.BlockSpec((2, _BLK, H), lambda i: (0, i, 0)),
            pl.BlockSpec((_BLK,), lambda i: (i,)),
            pl.BlockSpec((1, H), lambda i: (0, 0)),
        ],
        out_specs=pl.BlockSpec((_BLK, H), lambda i: (i, 0)),
        out_shape=jax.ShapeDtypeStruct((NP, H), jnp.float32),
    )(parts, nd, b2p)


def kernel(features, edge_index, W1, b1, W2, b2):
    src = edge_index[0]
    dst = edge_index[1]
    pad = jnp.full((EP - E,), N, dtype=jnp.int32)
    src_p = jnp.concatenate([src.astype(jnp.int32), pad]).reshape(R, CHUNK)
    dst_p = jnp.concatenate([dst.astype(jnp.int32), pad]).reshape(R, CHUNK)
    feat_p = jnp.pad(features, ((0, NP - N), (0, 0)))
    w2p = jnp.pad(W2, ((0, 0), (0, H - C)))
    b1r = b1.reshape(1, H)
    b2r = jnp.pad(b2, (0, H - C)).reshape(1, H)

    deg_o, deg_i = _deg_call(src_p, dst_p)
    h1s, ns, nd = _k1(feat_p, W1, deg_o, deg_i)
    agg1 = _gs_call(src_p, dst_p, h1s)
    h2s = _k2(agg1, nd, ns, b1r, w2p)
    agg2 = _gs_call(src_p, dst_p, h2s)
    out = _k3(agg2, nd, b2r)
    return out[:N, :C]
